# Initial kernel scaffold; baseline (speedup 1.0000x reference)
import functools

import jax
import jax.numpy as jnp
from jax import lax
from jax.experimental import pallas as pl
from jax.experimental.pallas import tpu as pltpu
from jax.experimental.pallas import tpu_sc as plsc

BATCH = 16384
HIST = 50
DIM = 32
NC = 2
NS = 16
NW = NC * NS
BAGS_PER_W = BATCH // NW
CHUNK = 64
NCHUNK = BAGS_PER_W // CHUNK
ROWS = CHUNK * HIST
SCALE = 1.0 / HIST


def _emb_bag_body(idx_hbm, table_hbm, out_hbm, idx_v, rows_v, out_v, sem):
    wid = lax.axis_index("s") * NC + lax.axis_index("c")
    bag_base = wid * BAGS_PER_W

    def chunk_body(c, carry):
        bag0 = bag_base + c * CHUNK
        pltpu.sync_copy(idx_hbm.at[pl.ds(bag0 * HIST, ROWS)], idx_v)
        pltpu.async_copy(table_hbm.at[idx_v], rows_v, sem).wait()

        def bag_body(i, carry2):
            r = i * HIST
            acc0 = rows_v[r, pl.ds(0, 16)]
            acc1 = rows_v[r, pl.ds(16, 16)]
            for j in range(1, HIST):
                acc0 = acc0 + rows_v[r + j, pl.ds(0, 16)]
                acc1 = acc1 + rows_v[r + j, pl.ds(16, 16)]
            out_v[i, pl.ds(0, 16)] = acc0 * SCALE
            out_v[i, pl.ds(16, 16)] = acc1 * SCALE
            return carry2

        lax.fori_loop(0, CHUNK, bag_body, 0)
        pltpu.sync_copy(out_v, out_hbm.at[pl.ds(bag0, CHUNK)])
        return carry

    lax.fori_loop(0, NCHUNK, chunk_body, 0)


def kernel(input, weight):
    idx = input.reshape(-1).astype(jnp.int32)
    mesh = plsc.VectorSubcoreMesh(core_axis_name="c", subcore_axis_name="s")
    run = functools.partial(
        pl.kernel,
        mesh=mesh,
        out_type=jax.ShapeDtypeStruct((BATCH, DIM), jnp.float32),
        scratch_types=[
            pltpu.VMEM((ROWS,), jnp.int32),
            pltpu.VMEM((ROWS, DIM), jnp.float32),
            pltpu.VMEM((CHUNK, DIM), jnp.float32),
            pltpu.SemaphoreType.DMA,
        ],
    )(_emb_bag_body)
    return run(idx, weight)


# speedup vs baseline: 2.8002x; 2.8002x over previous
import functools

import jax
import jax.numpy as jnp
from jax import lax
from jax.experimental import pallas as pl
from jax.experimental.pallas import tpu as pltpu
from jax.experimental.pallas import tpu_sc as plsc

BATCH = 16384
HIST = 50
DIM = 32
NC = 2
NS = 16
NW = NC * NS
BAGS_PER_W = BATCH // NW
CHUNK = 64
NCHUNK = BAGS_PER_W // CHUNK
ROWS = CHUNK * HIST
SCALE = 1.0 / HIST


def _emb_bag_body(idx_hbm, table_hbm, out_hbm, idx_v, rows_v, out_v, sem):
    wid = lax.axis_index("s") * NC + lax.axis_index("c")
    bag_base = wid * BAGS_PER_W

    def chunk_body(c, carry):
        bag0 = bag_base + c * CHUNK
        pltpu.sync_copy(idx_hbm.at[pl.ds(bag0 * HIST, ROWS)], idx_v)
        pltpu.async_copy(table_hbm.at[idx_v], rows_v, sem).wait()

        def bag_body(i, carry2):
            r = i * HIST
            acc0 = rows_v[r, pl.ds(0, 16)]
            acc1 = rows_v[r, pl.ds(16, 16)]
            for j in range(1, HIST):
                acc0 = acc0 + rows_v[r + j, pl.ds(0, 16)]
                acc1 = acc1 + rows_v[r + j, pl.ds(16, 16)]
            out_v[i, pl.ds(0, 16)] = acc0 * SCALE
            out_v[i, pl.ds(16, 16)] = acc1 * SCALE
            return carry2

        lax.fori_loop(0, CHUNK, bag_body, 0)
        pltpu.sync_copy(out_v, out_hbm.at[pl.ds(bag0, CHUNK)])
        return carry

    lax.fori_loop(0, NCHUNK, chunk_body, 0)


def kernel(input, weight):
    idx = input.reshape(-1).astype(jnp.int32)
    mesh = plsc.VectorSubcoreMesh(core_axis_name="c", subcore_axis_name="s")
    run = functools.partial(
        pl.kernel,
        mesh=mesh,
        compiler_params=pltpu.CompilerParams(use_tc_tiling_on_sc=False),
        out_type=jax.ShapeDtypeStruct((BATCH, DIM), jnp.float32),
        scratch_types=[
            pltpu.VMEM((ROWS,), jnp.int32),
            pltpu.VMEM((ROWS, DIM), jnp.float32),
            pltpu.VMEM((CHUNK, DIM), jnp.float32),
            pltpu.SemaphoreType.DMA,
        ],
    )(_emb_bag_body)
    return run(idx, weight)
